# Initial kernel scaffold; baseline (speedup 1.0000x reference)
import jax
import jax.numpy as jnp
from jax.experimental import pallas as pl


def kernel(preds, targets):
    raise NotImplementedError("write your pallas kernel here")


# speedup vs baseline: 15.5884x; 15.5884x over previous
import functools

import jax
import jax.numpy as jnp
from jax import lax
from jax.experimental import pallas as pl
from jax.experimental.pallas import tpu as pltpu
from jax.experimental.pallas import tpu_sc as plsc

IGNORE_LABEL = 255
THRESH = 0.35667494393873245

NC, NS, L = 2, 16, 16
NW = NC * NS

HB = 1024
INV_W = HB / THRESH
CH = 8192

N_PIX = 8 * 512 * 512
PER_W = N_PIX // NW
N_CHUNKS = PER_W // CH
N_MIN = float(max(N_PIX // 16, 1))


def _tc_loss_body(p_ref, t_ref, o_ref):
    t = t_ref[0]
    m = p_ref[0, 0]
    for c in range(1, 19):
        m = jnp.maximum(m, p_ref[0, c])
    s = jnp.zeros_like(m)
    xt = jnp.zeros_like(m)
    for c in range(19):
        xc = p_ref[0, c]
        s = s + jnp.exp(xc - m)
        xt = jnp.where(t == c, xc, xt)
    loss = m + jnp.log(s) - xt
    o_ref[0] = jnp.where(t == IGNORE_LABEL, 0.0, loss)


def _tc_loss(preds, targets):
    B, C, H, W = preds.shape
    RH = 128
    grid = (B, H // RH)
    return pl.pallas_call(
        _tc_loss_body,
        grid=grid,
        in_specs=[
            pl.BlockSpec((1, C, RH, W), lambda b, r: (b, 0, r, 0)),
            pl.BlockSpec((1, RH, W), lambda b, r: (b, r, 0)),
        ],
        out_specs=pl.BlockSpec((1, RH, W), lambda b, r: (b, r, 0)),
        out_shape=jax.ShapeDtypeStruct((B, H, W), jnp.float32),
        compiler_params=pltpu.CompilerParams(
            dimension_semantics=("parallel", "parallel"),
        ),
    )(preds, targets)


def _sc_mine_body(loss_hbm, cnt_out, sum_out, hard_out,
                  buf, cnts, sums, rcnt, rsum, hbuf):
    wid = lax.axis_index("s") * NC + lax.axis_index("c")
    base = wid * PER_W

    zero16 = jnp.zeros((L,), jnp.float32)

    def zbody(i, _):
        cnts[pl.ds(i * L, L)] = zero16
        sums[pl.ds(i * L, L)] = zero16
        return 0

    lax.fori_loop(0, HB, zbody, 0)

    lanes = lax.iota(jnp.int32, L) * HB
    ones = jnp.ones((L,), jnp.float32)

    def gbody(g, carry):
        hc, hs = carry
        v = buf[pl.ds(g * L, L)]
        hard = v > THRESH
        hc = hc + jnp.where(hard, 1.0, 0.0)
        hs = hs + jnp.where(hard, v, 0.0)
        b = (v * INV_W).astype(jnp.int32)
        b = jnp.minimum(b, HB - 1)
        fidx = lanes + b
        easy = jnp.logical_not(hard)
        plsc.addupdate_scatter(cnts, [fidx], ones, mask=easy)
        plsc.addupdate_scatter(sums, [fidx], v, mask=easy)
        return hc, hs

    hc, hs = zero16, zero16
    for chunk in range(N_CHUNKS):
        pltpu.sync_copy(loss_hbm.at[pl.ds(base + chunk * CH, CH)], buf)
        hc, hs = lax.fori_loop(0, CH // L, gbody, (hc, hs))

    def rbody(j, _):
        ac = zero16
        asm = zero16
        for l in range(L):
            ac = ac + cnts[pl.ds(l * HB + j * L, L)]
            asm = asm + sums[pl.ds(l * HB + j * L, L)]
        rcnt[pl.ds(j * L, L)] = ac
        rsum[pl.ds(j * L, L)] = asm
        return 0

    lax.fori_loop(0, HB // L, rbody, 0)

    hbuf[0] = hc
    hbuf[1] = hs
    pltpu.sync_copy(rcnt, cnt_out.at[wid])
    pltpu.sync_copy(rsum, sum_out.at[wid])
    pltpu.sync_copy(hbuf, hard_out.at[wid])


def _sc_mine(loss_flat):
    mesh = plsc.VectorSubcoreMesh(core_axis_name="c", subcore_axis_name="s")
    kern = pl.kernel(
        _sc_mine_body,
        mesh=mesh,
        out_type=[
            jax.ShapeDtypeStruct((NW, HB), jnp.float32),
            jax.ShapeDtypeStruct((NW, HB), jnp.float32),
            jax.ShapeDtypeStruct((NW, 2, L), jnp.float32),
        ],
        scratch_types=[
            pltpu.VMEM((CH,), jnp.float32),
            pltpu.VMEM((L * HB,), jnp.float32),
            pltpu.VMEM((L * HB,), jnp.float32),
            pltpu.VMEM((HB,), jnp.float32),
            pltpu.VMEM((HB,), jnp.float32),
            pltpu.VMEM((2, L), jnp.float32),
        ],
        compiler_params=pltpu.CompilerParams(needs_layout_passes=False),
    )
    return kern(loss_flat)


def _finish(cnt_hist, sum_hist, hard):
    cnt_b = jnp.sum(cnt_hist, axis=0)
    sum_b = jnp.sum(sum_hist, axis=0)
    h = jnp.sum(hard, axis=(0, 2))
    n_hard, sum_hard = h[0], h[1]
    hard_mean = sum_hard / jnp.maximum(n_hard, 1.0)
    cc = jnp.cumsum(cnt_b[::-1])[::-1]
    above = cc - cnt_b
    need = N_MIN - n_hard
    r = jnp.clip(need - above, 0.0, cnt_b)
    bin_mean = sum_b / jnp.maximum(cnt_b, 1.0)
    topk_mean = (sum_hard + jnp.sum(r * bin_mean)) / N_MIN
    return jnp.where(n_hard < N_MIN, topk_mean, hard_mean)


def kernel(preds, targets):
    loss = _tc_loss(preds, targets.astype(jnp.int32))
    cnt_hist, sum_hist, hard = _sc_mine(loss.reshape(-1))
    return _finish(cnt_hist, sum_hist, hard)


# speedup vs baseline: 23.7969x; 1.5266x over previous
import functools

import jax
import jax.numpy as jnp
from jax import lax
from jax.experimental import pallas as pl
from jax.experimental.pallas import tpu as pltpu
from jax.experimental.pallas import tpu_sc as plsc

IGNORE_LABEL = 255
THRESH = 0.35667494393873245

NC, NS, L = 2, 16, 16
NW = NC * NS

HB = 1024
INV_W = HB / THRESH
CH = 8192

N_PIX = 8 * 512 * 512
PER_W = N_PIX // NW
N_CHUNKS = PER_W // CH
N_MIN = float(max(N_PIX // 16, 1))


def _tc_loss_body(p_ref, t_ref, o_ref):
    t = t_ref[0]
    m = p_ref[0, 0]
    for c in range(1, 19):
        m = jnp.maximum(m, p_ref[0, c])
    s = jnp.zeros_like(m)
    xt = jnp.zeros_like(m)
    for c in range(19):
        xc = p_ref[0, c]
        s = s + jnp.exp(xc - m)
        xt = jnp.where(t == c, xc, xt)
    loss = m + jnp.log(s) - xt
    o_ref[0] = jnp.where(t == IGNORE_LABEL, 0.0, loss)


def _tc_loss(preds, targets):
    B, C, H, W = preds.shape
    RH = 128
    grid = (B, H // RH)
    return pl.pallas_call(
        _tc_loss_body,
        grid=grid,
        in_specs=[
            pl.BlockSpec((1, C, RH, W), lambda b, r: (b, 0, r, 0)),
            pl.BlockSpec((1, RH, W), lambda b, r: (b, r, 0)),
        ],
        out_specs=pl.BlockSpec((1, RH, W), lambda b, r: (b, r, 0)),
        out_shape=jax.ShapeDtypeStruct((B, H, W), jnp.float32),
        compiler_params=pltpu.CompilerParams(
            dimension_semantics=("parallel", "parallel"),
        ),
    )(preds, targets)


def _sc_hard_body(loss_hbm, hard_out, buf, obuf):
    wid = lax.axis_index("s") * NC + lax.axis_index("c")
    base = wid * PER_W

    def gbody(g, carry):
        hc, hs = carry
        for u in range(4):
            v = buf[pl.ds((g * 4 + u) * L, L)]
            hard = v > THRESH
            hc = hc + jnp.where(hard, 1.0, 0.0)
            hs = hs + jnp.where(hard, v, 0.0)
        return hc, hs

    hc = jnp.zeros((L,), jnp.float32)
    hs = jnp.zeros((L,), jnp.float32)
    for chunk in range(N_CHUNKS):
        pltpu.sync_copy(loss_hbm.at[pl.ds(base + chunk * CH, CH)], buf)
        hc, hs = lax.fori_loop(0, CH // (4 * L), gbody, (hc, hs))

    obuf[0] = hc
    obuf[1] = hs
    pltpu.sync_copy(obuf, hard_out.at[wid])


def _sc_hard(loss_flat):
    mesh = plsc.VectorSubcoreMesh(core_axis_name="c", subcore_axis_name="s")
    kern = pl.kernel(
        _sc_hard_body,
        mesh=mesh,
        out_type=jax.ShapeDtypeStruct((NW, 2, L), jnp.float32),
        scratch_types=[
            pltpu.VMEM((CH,), jnp.float32),
            pltpu.VMEM((2, L), jnp.float32),
        ],
        compiler_params=pltpu.CompilerParams(needs_layout_passes=False),
    )
    return kern(loss_flat)


def _sc_mine_body(loss_hbm, cnt_out, sum_out, hard_out,
                  buf, cnts, sums, rcnt, rsum, hbuf):
    wid = lax.axis_index("s") * NC + lax.axis_index("c")
    base = wid * PER_W

    zero16 = jnp.zeros((L,), jnp.float32)

    def zbody(i, _):
        cnts[pl.ds(i * L, L)] = zero16
        sums[pl.ds(i * L, L)] = zero16
        return 0

    lax.fori_loop(0, HB, zbody, 0)

    lanes = lax.iota(jnp.int32, L) * HB
    ones = jnp.ones((L,), jnp.float32)

    def gbody(g, carry):
        hc, hs = carry
        v = buf[pl.ds(g * L, L)]
        hard = v > THRESH
        hc = hc + jnp.where(hard, 1.0, 0.0)
        hs = hs + jnp.where(hard, v, 0.0)
        b = (v * INV_W).astype(jnp.int32)
        b = jnp.minimum(b, HB - 1)
        fidx = lanes + b
        easy = jnp.logical_not(hard)
        plsc.addupdate_scatter(cnts, [fidx], ones, mask=easy)
        plsc.addupdate_scatter(sums, [fidx], v, mask=easy)
        return hc, hs

    hc, hs = zero16, zero16
    for chunk in range(N_CHUNKS):
        pltpu.sync_copy(loss_hbm.at[pl.ds(base + chunk * CH, CH)], buf)
        hc, hs = lax.fori_loop(0, CH // L, gbody, (hc, hs))

    def rbody(j, _):
        ac = zero16
        asm = zero16
        for l in range(L):
            ac = ac + cnts[pl.ds(l * HB + j * L, L)]
            asm = asm + sums[pl.ds(l * HB + j * L, L)]
        rcnt[pl.ds(j * L, L)] = ac
        rsum[pl.ds(j * L, L)] = asm
        return 0

    lax.fori_loop(0, HB // L, rbody, 0)

    hbuf[0] = hc
    hbuf[1] = hs
    pltpu.sync_copy(rcnt, cnt_out.at[wid])
    pltpu.sync_copy(rsum, sum_out.at[wid])
    pltpu.sync_copy(hbuf, hard_out.at[wid])


def _sc_mine(loss_flat):
    mesh = plsc.VectorSubcoreMesh(core_axis_name="c", subcore_axis_name="s")
    kern = pl.kernel(
        _sc_mine_body,
        mesh=mesh,
        out_type=[
            jax.ShapeDtypeStruct((NW, HB), jnp.float32),
            jax.ShapeDtypeStruct((NW, HB), jnp.float32),
            jax.ShapeDtypeStruct((NW, 2, L), jnp.float32),
        ],
        scratch_types=[
            pltpu.VMEM((CH,), jnp.float32),
            pltpu.VMEM((L * HB,), jnp.float32),
            pltpu.VMEM((L * HB,), jnp.float32),
            pltpu.VMEM((HB,), jnp.float32),
            pltpu.VMEM((HB,), jnp.float32),
            pltpu.VMEM((2, L), jnp.float32),
        ],
        compiler_params=pltpu.CompilerParams(needs_layout_passes=False),
    )
    return kern(loss_flat)


def _topk_mean(cnt_hist, sum_hist, hard):
    cnt_b = jnp.sum(cnt_hist, axis=0)
    sum_b = jnp.sum(sum_hist, axis=0)
    h = jnp.sum(hard, axis=(0, 2))
    n_hard, sum_hard = h[0], h[1]
    cc = jnp.cumsum(cnt_b[::-1])[::-1]
    above = cc - cnt_b
    need = N_MIN - n_hard
    r = jnp.clip(need - above, 0.0, cnt_b)
    bin_mean = sum_b / jnp.maximum(cnt_b, 1.0)
    return (sum_hard + jnp.sum(r * bin_mean)) / N_MIN


def _finish(cnt_hist, sum_hist, hard):
    h = jnp.sum(hard, axis=(0, 2))
    n_hard = h[0]
    hard_mean = h[1] / jnp.maximum(n_hard, 1.0)
    return jnp.where(n_hard < N_MIN, _topk_mean(cnt_hist, sum_hist, hard),
                     hard_mean)


def kernel(preds, targets):
    loss = _tc_loss(preds, targets.astype(jnp.int32))
    loss_flat = loss.reshape(-1)
    hstats = _sc_hard(loss_flat)
    h = jnp.sum(hstats, axis=(0, 2))
    n_hard, sum_hard = h[0], h[1]
    hard_mean = sum_hard / jnp.maximum(n_hard, 1.0)

    def rare(_):
        cnt_hist, sum_hist, hard = _sc_mine(loss_flat)
        return _topk_mean(cnt_hist, sum_hist, hard)

    def common(_):
        return hard_mean

    return lax.cond(n_hard < N_MIN, rare, common, None)


# speedup vs baseline: 26.5249x; 1.1146x over previous
import functools

import jax
import jax.numpy as jnp
from jax import lax
from jax.experimental import pallas as pl
from jax.experimental.pallas import tpu as pltpu
from jax.experimental.pallas import tpu_sc as plsc

IGNORE_LABEL = 255
THRESH = 0.35667494393873245

NC, NS, L = 2, 16, 16
NW = NC * NS

HB = 1024
INV_W = HB / THRESH
CH = 8192

N_PIX = 8 * 512 * 512
PER_W = N_PIX // NW
N_CHUNKS = PER_W // CH
N_MIN = float(max(N_PIX // 16, 1))


def _tc_loss_body(p_ref, t_ref, o_ref):
    t = t_ref[0]
    m = p_ref[0, 0]
    for c in range(1, 19):
        m = jnp.maximum(m, p_ref[0, c])
    s = jnp.zeros_like(m)
    xt = jnp.zeros_like(m)
    for c in range(19):
        xc = p_ref[0, c]
        s = s + jnp.exp(xc - m)
        xt = jnp.where(t == c, xc, xt)
    loss = m + jnp.log(s) - xt
    o_ref[...] = jnp.where(t == IGNORE_LABEL, 0.0, loss).reshape(-1)


def _tc_loss(preds, targets):
    B, C, H, W = preds.shape
    RH = 128
    NR = H // RH
    grid = (B, NR)
    return pl.pallas_call(
        _tc_loss_body,
        grid=grid,
        in_specs=[
            pl.BlockSpec((1, C, RH, W), lambda b, r: (b, 0, r, 0)),
            pl.BlockSpec((1, RH, W), lambda b, r: (b, r, 0)),
        ],
        out_specs=pl.BlockSpec((RH * W,), lambda b, r: (b * NR + r,)),
        out_shape=jax.ShapeDtypeStruct((B * H * W,), jnp.float32),
        compiler_params=pltpu.CompilerParams(
            dimension_semantics=("parallel", "parallel"),
        ),
    )(preds, targets)


def _sc_hard_body(loss_hbm, hard_out, buf, obuf):
    wid = lax.axis_index("s") * NC + lax.axis_index("c")
    base = wid * PER_W

    def gbody(g, carry):
        hc, hs = carry
        for u in range(4):
            v = buf[pl.ds((g * 4 + u) * L, L)]
            hard = v > THRESH
            hc = hc + jnp.where(hard, 1.0, 0.0)
            hs = hs + jnp.where(hard, v, 0.0)
        return hc, hs

    hc = jnp.zeros((L,), jnp.float32)
    hs = jnp.zeros((L,), jnp.float32)
    for chunk in range(N_CHUNKS):
        pltpu.sync_copy(loss_hbm.at[pl.ds(base + chunk * CH, CH)], buf)
        hc, hs = lax.fori_loop(0, CH // (4 * L), gbody, (hc, hs))

    obuf[0] = hc
    obuf[1] = hs
    pltpu.sync_copy(obuf, hard_out.at[wid])


def _sc_hard(loss_flat):
    mesh = plsc.VectorSubcoreMesh(core_axis_name="c", subcore_axis_name="s")
    kern = pl.kernel(
        _sc_hard_body,
        mesh=mesh,
        out_type=jax.ShapeDtypeStruct((NW, 2, L), jnp.float32),
        scratch_types=[
            pltpu.VMEM((CH,), jnp.float32),
            pltpu.VMEM((2, L), jnp.float32),
        ],
        compiler_params=pltpu.CompilerParams(needs_layout_passes=False),
    )
    return kern(loss_flat)


def _sc_mine_body(loss_hbm, cnt_out, sum_out, hard_out,
                  buf, cnts, sums, rcnt, rsum, hbuf):
    wid = lax.axis_index("s") * NC + lax.axis_index("c")
    base = wid * PER_W

    zero16 = jnp.zeros((L,), jnp.float32)

    def zbody(i, _):
        cnts[pl.ds(i * L, L)] = zero16
        sums[pl.ds(i * L, L)] = zero16
        return 0

    lax.fori_loop(0, HB, zbody, 0)

    lanes = lax.iota(jnp.int32, L) * HB
    ones = jnp.ones((L,), jnp.float32)

    def gbody(g, carry):
        hc, hs = carry
        v = buf[pl.ds(g * L, L)]
        hard = v > THRESH
        hc = hc + jnp.where(hard, 1.0, 0.0)
        hs = hs + jnp.where(hard, v, 0.0)
        b = (v * INV_W).astype(jnp.int32)
        b = jnp.minimum(b, HB - 1)
        fidx = lanes + b
        easy = jnp.logical_not(hard)
        plsc.addupdate_scatter(cnts, [fidx], ones, mask=easy)
        plsc.addupdate_scatter(sums, [fidx], v, mask=easy)
        return hc, hs

    hc, hs = zero16, zero16
    for chunk in range(N_CHUNKS):
        pltpu.sync_copy(loss_hbm.at[pl.ds(base + chunk * CH, CH)], buf)
        hc, hs = lax.fori_loop(0, CH // L, gbody, (hc, hs))

    def rbody(j, _):
        ac = zero16
        asm = zero16
        for l in range(L):
            ac = ac + cnts[pl.ds(l * HB + j * L, L)]
            asm = asm + sums[pl.ds(l * HB + j * L, L)]
        rcnt[pl.ds(j * L, L)] = ac
        rsum[pl.ds(j * L, L)] = asm
        return 0

    lax.fori_loop(0, HB // L, rbody, 0)

    hbuf[0] = hc
    hbuf[1] = hs
    pltpu.sync_copy(rcnt, cnt_out.at[wid])
    pltpu.sync_copy(rsum, sum_out.at[wid])
    pltpu.sync_copy(hbuf, hard_out.at[wid])


def _sc_mine(loss_flat):
    mesh = plsc.VectorSubcoreMesh(core_axis_name="c", subcore_axis_name="s")
    kern = pl.kernel(
        _sc_mine_body,
        mesh=mesh,
        out_type=[
            jax.ShapeDtypeStruct((NW, HB), jnp.float32),
            jax.ShapeDtypeStruct((NW, HB), jnp.float32),
            jax.ShapeDtypeStruct((NW, 2, L), jnp.float32),
        ],
        scratch_types=[
            pltpu.VMEM((CH,), jnp.float32),
            pltpu.VMEM((L * HB,), jnp.float32),
            pltpu.VMEM((L * HB,), jnp.float32),
            pltpu.VMEM((HB,), jnp.float32),
            pltpu.VMEM((HB,), jnp.float32),
            pltpu.VMEM((2, L), jnp.float32),
        ],
        compiler_params=pltpu.CompilerParams(needs_layout_passes=False),
    )
    return kern(loss_flat)


def _topk_mean(cnt_hist, sum_hist, hard):
    cnt_b = jnp.sum(cnt_hist, axis=0)
    sum_b = jnp.sum(sum_hist, axis=0)
    h = jnp.sum(hard, axis=(0, 2))
    n_hard, sum_hard = h[0], h[1]
    cc = jnp.cumsum(cnt_b[::-1])[::-1]
    above = cc - cnt_b
    need = N_MIN - n_hard
    r = jnp.clip(need - above, 0.0, cnt_b)
    bin_mean = sum_b / jnp.maximum(cnt_b, 1.0)
    return (sum_hard + jnp.sum(r * bin_mean)) / N_MIN


def _finish(cnt_hist, sum_hist, hard):
    h = jnp.sum(hard, axis=(0, 2))
    n_hard = h[0]
    hard_mean = h[1] / jnp.maximum(n_hard, 1.0)
    return jnp.where(n_hard < N_MIN, _topk_mean(cnt_hist, sum_hist, hard),
                     hard_mean)


def kernel(preds, targets):
    loss_flat = _tc_loss(preds, targets.astype(jnp.int32))
    hstats = _sc_hard(loss_flat)
    h = jnp.sum(hstats, axis=(0, 2))
    n_hard, sum_hard = h[0], h[1]
    hard_mean = sum_hard / jnp.maximum(n_hard, 1.0)

    def rare(_):
        cnt_hist, sum_hist, hard = _sc_mine(loss_flat)
        return _topk_mean(cnt_hist, sum_hist, hard)

    def common(_):
        return hard_mean

    return lax.cond(n_hard < N_MIN, rare, common, None)


# speedup vs baseline: 28.6363x; 1.0796x over previous
import functools

import jax
import jax.numpy as jnp
from jax import lax
from jax.experimental import pallas as pl
from jax.experimental.pallas import tpu as pltpu
from jax.experimental.pallas import tpu_sc as plsc

IGNORE_LABEL = 255
THRESH = 0.35667494393873245

NC, NS, L = 2, 16, 16
NW = NC * NS

HB = 1024
INV_W = HB / THRESH
CH = 8192

N_PIX = 8 * 512 * 512
PER_W = N_PIX // NW
N_CHUNKS = PER_W // CH
N_MIN = float(max(N_PIX // 16, 1))


def _tc_loss_body(p_ref, t_ref, o_ref):
    t = t_ref[0]
    m = p_ref[0, 0]
    for c in range(1, 19):
        m = jnp.maximum(m, p_ref[0, c])
    s = jnp.zeros_like(m)
    xt = jnp.zeros_like(m)
    for c in range(19):
        xc = p_ref[0, c]
        s = s + jnp.exp(xc - m)
        xt = jnp.where(t == c, xc, xt)
    loss = m + jnp.log(s) - xt
    o_ref[...] = jnp.where(t == IGNORE_LABEL, 0.0, loss).reshape(-1)


def _tc_loss(preds, targets):
    B, C, H, W = preds.shape
    RH = 256
    NR = H // RH
    grid = (B, NR)
    return pl.pallas_call(
        _tc_loss_body,
        grid=grid,
        in_specs=[
            pl.BlockSpec((1, C, RH, W), lambda b, r: (b, 0, r, 0)),
            pl.BlockSpec((1, RH, W), lambda b, r: (b, r, 0)),
        ],
        out_specs=pl.BlockSpec((RH * W,), lambda b, r: (b * NR + r,)),
        out_shape=jax.ShapeDtypeStruct((B * H * W,), jnp.float32),
        compiler_params=pltpu.CompilerParams(
            dimension_semantics=("parallel", "parallel"),
        ),
    )(preds, targets)


def _sc_hard_body(loss_hbm, hard_out, buf, obuf):
    wid = lax.axis_index("s") * NC + lax.axis_index("c")
    base = wid * PER_W

    def gbody(g, carry):
        hc, hs = carry
        for u in range(4):
            v = buf[pl.ds((g * 4 + u) * L, L)]
            hard = v > THRESH
            hc = hc + jnp.where(hard, 1.0, 0.0)
            hs = hs + jnp.where(hard, v, 0.0)
        return hc, hs

    hc = jnp.zeros((L,), jnp.float32)
    hs = jnp.zeros((L,), jnp.float32)
    for chunk in range(N_CHUNKS):
        pltpu.sync_copy(loss_hbm.at[pl.ds(base + chunk * CH, CH)], buf)
        hc, hs = lax.fori_loop(0, CH // (4 * L), gbody, (hc, hs))

    obuf[0] = hc
    obuf[1] = hs
    pltpu.sync_copy(obuf, hard_out.at[wid])


def _sc_hard(loss_flat):
    mesh = plsc.VectorSubcoreMesh(core_axis_name="c", subcore_axis_name="s")
    kern = pl.kernel(
        _sc_hard_body,
        mesh=mesh,
        out_type=jax.ShapeDtypeStruct((NW, 2, L), jnp.float32),
        scratch_types=[
            pltpu.VMEM((CH,), jnp.float32),
            pltpu.VMEM((2, L), jnp.float32),
        ],
        compiler_params=pltpu.CompilerParams(needs_layout_passes=False),
    )
    return kern(loss_flat)


def _sc_mine_body(loss_hbm, cnt_out, sum_out, hard_out,
                  buf, cnts, sums, rcnt, rsum, hbuf):
    wid = lax.axis_index("s") * NC + lax.axis_index("c")
    base = wid * PER_W

    zero16 = jnp.zeros((L,), jnp.float32)

    def zbody(i, _):
        cnts[pl.ds(i * L, L)] = zero16
        sums[pl.ds(i * L, L)] = zero16
        return 0

    lax.fori_loop(0, HB, zbody, 0)

    lanes = lax.iota(jnp.int32, L) * HB
    ones = jnp.ones((L,), jnp.float32)

    def gbody(g, carry):
        hc, hs = carry
        v = buf[pl.ds(g * L, L)]
        hard = v > THRESH
        hc = hc + jnp.where(hard, 1.0, 0.0)
        hs = hs + jnp.where(hard, v, 0.0)
        b = (v * INV_W).astype(jnp.int32)
        b = jnp.minimum(b, HB - 1)
        fidx = lanes + b
        easy = jnp.logical_not(hard)
        plsc.addupdate_scatter(cnts, [fidx], ones, mask=easy)
        plsc.addupdate_scatter(sums, [fidx], v, mask=easy)
        return hc, hs

    hc, hs = zero16, zero16
    for chunk in range(N_CHUNKS):
        pltpu.sync_copy(loss_hbm.at[pl.ds(base + chunk * CH, CH)], buf)
        hc, hs = lax.fori_loop(0, CH // L, gbody, (hc, hs))

    def rbody(j, _):
        ac = zero16
        asm = zero16
        for l in range(L):
            ac = ac + cnts[pl.ds(l * HB + j * L, L)]
            asm = asm + sums[pl.ds(l * HB + j * L, L)]
        rcnt[pl.ds(j * L, L)] = ac
        rsum[pl.ds(j * L, L)] = asm
        return 0

    lax.fori_loop(0, HB // L, rbody, 0)

    hbuf[0] = hc
    hbuf[1] = hs
    pltpu.sync_copy(rcnt, cnt_out.at[wid])
    pltpu.sync_copy(rsum, sum_out.at[wid])
    pltpu.sync_copy(hbuf, hard_out.at[wid])


def _sc_mine(loss_flat):
    mesh = plsc.VectorSubcoreMesh(core_axis_name="c", subcore_axis_name="s")
    kern = pl.kernel(
        _sc_mine_body,
        mesh=mesh,
        out_type=[
            jax.ShapeDtypeStruct((NW, HB), jnp.float32),
            jax.ShapeDtypeStruct((NW, HB), jnp.float32),
            jax.ShapeDtypeStruct((NW, 2, L), jnp.float32),
        ],
        scratch_types=[
            pltpu.VMEM((CH,), jnp.float32),
            pltpu.VMEM((L * HB,), jnp.float32),
            pltpu.VMEM((L * HB,), jnp.float32),
            pltpu.VMEM((HB,), jnp.float32),
            pltpu.VMEM((HB,), jnp.float32),
            pltpu.VMEM((2, L), jnp.float32),
        ],
        compiler_params=pltpu.CompilerParams(needs_layout_passes=False),
    )
    return kern(loss_flat)


def _topk_mean(cnt_hist, sum_hist, hard):
    cnt_b = jnp.sum(cnt_hist, axis=0)
    sum_b = jnp.sum(sum_hist, axis=0)
    h = jnp.sum(hard, axis=(0, 2))
    n_hard, sum_hard = h[0], h[1]
    cc = jnp.cumsum(cnt_b[::-1])[::-1]
    above = cc - cnt_b
    need = N_MIN - n_hard
    r = jnp.clip(need - above, 0.0, cnt_b)
    bin_mean = sum_b / jnp.maximum(cnt_b, 1.0)
    return (sum_hard + jnp.sum(r * bin_mean)) / N_MIN


def _finish(cnt_hist, sum_hist, hard):
    h = jnp.sum(hard, axis=(0, 2))
    n_hard = h[0]
    hard_mean = h[1] / jnp.maximum(n_hard, 1.0)
    return jnp.where(n_hard < N_MIN, _topk_mean(cnt_hist, sum_hist, hard),
                     hard_mean)


def kernel(preds, targets):
    loss_flat = _tc_loss(preds, targets.astype(jnp.int32))
    hstats = _sc_hard(loss_flat)
    h = jnp.sum(hstats, axis=(0, 2))
    n_hard, sum_hard = h[0], h[1]
    hard_mean = sum_hard / jnp.maximum(n_hard, 1.0)

    def rare(_):
        cnt_hist, sum_hist, hard = _sc_mine(loss_flat)
        return _topk_mean(cnt_hist, sum_hist, hard)

    def common(_):
        return hard_mean

    return lax.cond(n_hard < N_MIN, rare, common, None)


# speedup vs baseline: 28.7871x; 1.0053x over previous
import functools

import jax
import jax.numpy as jnp
from jax import lax
from jax.experimental import pallas as pl
from jax.experimental.pallas import tpu as pltpu
from jax.experimental.pallas import tpu_sc as plsc

IGNORE_LABEL = 255
THRESH = 0.35667494393873245

NC, NS, L = 2, 16, 16
NW = NC * NS

HB = 1024
INV_W = HB / THRESH
CH = 8192

N_PIX = 8 * 512 * 512
PER_W = N_PIX // NW
N_CHUNKS = PER_W // CH
N_MIN = float(max(N_PIX // 16, 1))


def _tc_loss_body(p_ref, t_ref, o_ref):
    t = t_ref[0]
    m = p_ref[0, 0]
    for c in range(1, 19):
        m = jnp.maximum(m, p_ref[0, c])
    s = jnp.zeros_like(m)
    xt = jnp.zeros_like(m)
    for c in range(19):
        xc = p_ref[0, c]
        s = s + jnp.exp(xc - m)
        xt = jnp.where(t == c, xc, xt)
    loss = m + jnp.log(s) - xt
    o_ref[...] = jnp.where(t == IGNORE_LABEL, 0.0, loss).reshape(-1)


def _tc_loss(preds, targets):
    B, C, H, W = preds.shape
    RH = 512
    NR = H // RH
    grid = (B, NR)
    return pl.pallas_call(
        _tc_loss_body,
        grid=grid,
        in_specs=[
            pl.BlockSpec((1, C, RH, W), lambda b, r: (b, 0, r, 0)),
            pl.BlockSpec((1, RH, W), lambda b, r: (b, r, 0)),
        ],
        out_specs=pl.BlockSpec((RH * W,), lambda b, r: (b * NR + r,)),
        out_shape=jax.ShapeDtypeStruct((B * H * W,), jnp.float32),
        compiler_params=pltpu.CompilerParams(
            dimension_semantics=("parallel", "parallel"),
        ),
    )(preds, targets)


def _sc_hard_body(loss_hbm, hard_out, buf, obuf):
    wid = lax.axis_index("s") * NC + lax.axis_index("c")
    base = wid * PER_W

    def gbody(g, carry):
        hc, hs = carry
        for u in range(4):
            v = buf[pl.ds((g * 4 + u) * L, L)]
            hard = v > THRESH
            hc = hc + jnp.where(hard, 1.0, 0.0)
            hs = hs + jnp.where(hard, v, 0.0)
        return hc, hs

    hc = jnp.zeros((L,), jnp.float32)
    hs = jnp.zeros((L,), jnp.float32)
    for chunk in range(N_CHUNKS):
        pltpu.sync_copy(loss_hbm.at[pl.ds(base + chunk * CH, CH)], buf)
        hc, hs = lax.fori_loop(0, CH // (4 * L), gbody, (hc, hs))

    obuf[0] = hc
    obuf[1] = hs
    pltpu.sync_copy(obuf, hard_out.at[wid])


def _sc_hard(loss_flat):
    mesh = plsc.VectorSubcoreMesh(core_axis_name="c", subcore_axis_name="s")
    kern = pl.kernel(
        _sc_hard_body,
        mesh=mesh,
        out_type=jax.ShapeDtypeStruct((NW, 2, L), jnp.float32),
        scratch_types=[
            pltpu.VMEM((CH,), jnp.float32),
            pltpu.VMEM((2, L), jnp.float32),
        ],
        compiler_params=pltpu.CompilerParams(needs_layout_passes=False),
    )
    return kern(loss_flat)


def _sc_mine_body(loss_hbm, cnt_out, sum_out, hard_out,
                  buf, cnts, sums, rcnt, rsum, hbuf):
    wid = lax.axis_index("s") * NC + lax.axis_index("c")
    base = wid * PER_W

    zero16 = jnp.zeros((L,), jnp.float32)

    def zbody(i, _):
        cnts[pl.ds(i * L, L)] = zero16
        sums[pl.ds(i * L, L)] = zero16
        return 0

    lax.fori_loop(0, HB, zbody, 0)

    lanes = lax.iota(jnp.int32, L) * HB
    ones = jnp.ones((L,), jnp.float32)

    def gbody(g, carry):
        hc, hs = carry
        v = buf[pl.ds(g * L, L)]
        hard = v > THRESH
        hc = hc + jnp.where(hard, 1.0, 0.0)
        hs = hs + jnp.where(hard, v, 0.0)
        b = (v * INV_W).astype(jnp.int32)
        b = jnp.minimum(b, HB - 1)
        fidx = lanes + b
        easy = jnp.logical_not(hard)
        plsc.addupdate_scatter(cnts, [fidx], ones, mask=easy)
        plsc.addupdate_scatter(sums, [fidx], v, mask=easy)
        return hc, hs

    hc, hs = zero16, zero16
    for chunk in range(N_CHUNKS):
        pltpu.sync_copy(loss_hbm.at[pl.ds(base + chunk * CH, CH)], buf)
        hc, hs = lax.fori_loop(0, CH // L, gbody, (hc, hs))

    def rbody(j, _):
        ac = zero16
        asm = zero16
        for l in range(L):
            ac = ac + cnts[pl.ds(l * HB + j * L, L)]
            asm = asm + sums[pl.ds(l * HB + j * L, L)]
        rcnt[pl.ds(j * L, L)] = ac
        rsum[pl.ds(j * L, L)] = asm
        return 0

    lax.fori_loop(0, HB // L, rbody, 0)

    hbuf[0] = hc
    hbuf[1] = hs
    pltpu.sync_copy(rcnt, cnt_out.at[wid])
    pltpu.sync_copy(rsum, sum_out.at[wid])
    pltpu.sync_copy(hbuf, hard_out.at[wid])


def _sc_mine(loss_flat):
    mesh = plsc.VectorSubcoreMesh(core_axis_name="c", subcore_axis_name="s")
    kern = pl.kernel(
        _sc_mine_body,
        mesh=mesh,
        out_type=[
            jax.ShapeDtypeStruct((NW, HB), jnp.float32),
            jax.ShapeDtypeStruct((NW, HB), jnp.float32),
            jax.ShapeDtypeStruct((NW, 2, L), jnp.float32),
        ],
        scratch_types=[
            pltpu.VMEM((CH,), jnp.float32),
            pltpu.VMEM((L * HB,), jnp.float32),
            pltpu.VMEM((L * HB,), jnp.float32),
            pltpu.VMEM((HB,), jnp.float32),
            pltpu.VMEM((HB,), jnp.float32),
            pltpu.VMEM((2, L), jnp.float32),
        ],
        compiler_params=pltpu.CompilerParams(needs_layout_passes=False),
    )
    return kern(loss_flat)


def _topk_mean(cnt_hist, sum_hist, hard):
    cnt_b = jnp.sum(cnt_hist, axis=0)
    sum_b = jnp.sum(sum_hist, axis=0)
    h = jnp.sum(hard, axis=(0, 2))
    n_hard, sum_hard = h[0], h[1]
    cc = jnp.cumsum(cnt_b[::-1])[::-1]
    above = cc - cnt_b
    need = N_MIN - n_hard
    r = jnp.clip(need - above, 0.0, cnt_b)
    bin_mean = sum_b / jnp.maximum(cnt_b, 1.0)
    return (sum_hard + jnp.sum(r * bin_mean)) / N_MIN


def _finish(cnt_hist, sum_hist, hard):
    h = jnp.sum(hard, axis=(0, 2))
    n_hard = h[0]
    hard_mean = h[1] / jnp.maximum(n_hard, 1.0)
    return jnp.where(n_hard < N_MIN, _topk_mean(cnt_hist, sum_hist, hard),
                     hard_mean)


def kernel(preds, targets):
    loss_flat = _tc_loss(preds, targets.astype(jnp.int32))
    hstats = _sc_hard(loss_flat)
    h = jnp.sum(hstats, axis=(0, 2))
    n_hard, sum_hard = h[0], h[1]
    hard_mean = sum_hard / jnp.maximum(n_hard, 1.0)

    def rare(_):
        cnt_hist, sum_hist, hard = _sc_mine(loss_flat)
        return _topk_mean(cnt_hist, sum_hist, hard)

    def common(_):
        return hard_mean

    return lax.cond(n_hard < N_MIN, rare, common, None)
